# Initial kernel scaffold; baseline (speedup 1.0000x reference)
import jax
import jax.numpy as jnp
from jax.experimental import pallas as pl


def kernel(indices, graph_x, kernel, W0, W1, Wlin):
    raise NotImplementedError("write your pallas kernel here")


# speedup vs baseline: 28.4548x; 28.4548x over previous
import jax
import jax.numpy as jnp
from jax.experimental import pallas as pl
from jax.experimental.pallas import tpu as pltpu

NODES = 26
IN_DIM = 128
H0 = 48
H1 = 60


def _graph_kernel(idx_ref, x_ref, k_ref, w0_ref, w1_ref, wlin_ref, out_ref):
    x = x_ref[0]
    k = k_ref[0]
    dinv_c = jax.lax.rsqrt(jnp.sum(k, axis=0, keepdims=True))
    dinv_r = jax.lax.rsqrt(jnp.sum(k, axis=1, keepdims=True))
    a = k * dinv_r * dinv_c
    m1 = jnp.dot(x, w0_ref[...], preferred_element_type=jnp.float32)
    h = jnp.maximum(jnp.dot(a, m1, preferred_element_type=jnp.float32,
                            precision=jax.lax.Precision.HIGHEST), 0.0)
    m2 = jnp.dot(h, w1_ref[...], preferred_element_type=jnp.float32)
    z = jnp.maximum(jnp.dot(a[0:1, :], m2, preferred_element_type=jnp.float32,
                            precision=jax.lax.Precision.HIGHEST), 0.0)
    p = jnp.maximum(jnp.dot(z, wlin_ref[...],
                            preferred_element_type=jnp.float32), 0.0)
    out_ref[0] = p


def kernel(indices, graph_x, kernel, W0, W1, Wlin):
    B = indices.shape[0]
    grid_spec = pltpu.PrefetchScalarGridSpec(
        num_scalar_prefetch=1,
        grid=(B,),
        in_specs=[
            pl.BlockSpec((1, NODES, IN_DIM), lambda i, idx: (idx[i], 0, 0)),
            pl.BlockSpec((1, NODES, NODES), lambda i, idx: (idx[i], 0, 0)),
            pl.BlockSpec((IN_DIM, H0), lambda i, idx: (0, 0)),
            pl.BlockSpec((H0, H1), lambda i, idx: (0, 0)),
            pl.BlockSpec((H1, 1), lambda i, idx: (0, 0)),
        ],
        out_specs=pl.BlockSpec((1, 1, 1), lambda i, idx: (i, 0, 0)),
    )
    out = pl.pallas_call(
        _graph_kernel,
        grid_spec=grid_spec,
        out_shape=jax.ShapeDtypeStruct((B, 1, 1), jnp.float32),
    )(indices, graph_x, kernel, W0, W1, Wlin)
    return out.reshape(B, 1)


# speedup vs baseline: 35.8086x; 1.2584x over previous
import jax
import jax.numpy as jnp
from jax.experimental import pallas as pl
from jax.experimental.pallas import tpu as pltpu

NODES = 26
IN_DIM = 128
H0 = 48
H1 = 60
G = 8


def _graph_kernel(idx_ref, *refs):
    x_refs = refs[:G]
    k_refs = refs[G:2 * G]
    w0_ref, w1_ref, wlin_ref, out_ref = refs[2 * G:]
    for g in range(G):
        x = x_refs[g][0]
        k = k_refs[g][0]
        dinv_c = jax.lax.rsqrt(jnp.sum(k, axis=0, keepdims=True))
        dinv_r = jax.lax.rsqrt(jnp.sum(k, axis=1, keepdims=True))
        a = k * dinv_r * dinv_c
        m1 = jnp.dot(x, w0_ref[...], preferred_element_type=jnp.float32)
        h = jnp.maximum(jnp.dot(a, m1, preferred_element_type=jnp.float32,
                                precision=jax.lax.Precision.HIGHEST), 0.0)
        m2 = jnp.dot(h, w1_ref[...], preferred_element_type=jnp.float32)
        z = jnp.maximum(jnp.dot(a[0:1, :], m2,
                                preferred_element_type=jnp.float32,
                                precision=jax.lax.Precision.HIGHEST), 0.0)
        p = jnp.maximum(jnp.dot(z, wlin_ref[...],
                                preferred_element_type=jnp.float32), 0.0)
        out_ref[g] = p


def kernel(indices, graph_x, kernel, W0, W1, Wlin):
    B = indices.shape[0]
    x_specs = [
        pl.BlockSpec((1, NODES, IN_DIM),
                     (lambda i, idx, g=g: (idx[i * G + g], 0, 0)))
        for g in range(G)
    ]
    k_specs = [
        pl.BlockSpec((1, NODES, NODES),
                     (lambda i, idx, g=g: (idx[i * G + g], 0, 0)))
        for g in range(G)
    ]
    grid_spec = pltpu.PrefetchScalarGridSpec(
        num_scalar_prefetch=1,
        grid=(B // G,),
        in_specs=x_specs + k_specs + [
            pl.BlockSpec((IN_DIM, H0), lambda i, idx: (0, 0)),
            pl.BlockSpec((H0, H1), lambda i, idx: (0, 0)),
            pl.BlockSpec((H1, 1), lambda i, idx: (0, 0)),
        ],
        out_specs=pl.BlockSpec((G, 1, 1), lambda i, idx: (i, 0, 0)),
    )
    out = pl.pallas_call(
        _graph_kernel,
        grid_spec=grid_spec,
        out_shape=jax.ShapeDtypeStruct((B, 1, 1), jnp.float32),
    )(indices, *([graph_x] * G), *([kernel] * G), W0, W1, Wlin)
    return out.reshape(B, 1)


# speedup vs baseline: 90.4289x; 2.5253x over previous
import jax
import jax.numpy as jnp
from jax.experimental import pallas as pl
from jax.experimental.pallas import tpu as pltpu

NODES = 26
P = 32
IN_DIM = 128
H0 = 48
H1 = 60
G = 8
GP = G * P


def _graph_kernel(idx_ref, *refs):
    x_refs = refs[:G]
    k_refs = refs[G:2 * G]
    w0_ref, w1_ref, wlin_ref, out_ref, kbd, m1s, dall, asel = refs[2 * G:]

    @pl.when(pl.program_id(0) == 0)
    def _init():
        kbd[...] = jnp.zeros((GP, GP), jnp.float32)
        m1s[...] = jnp.zeros((GP, H0), jnp.float32)
        dall[...] = jnp.zeros((GP, 1), jnp.float32)
        asel[...] = jnp.zeros((G, GP), jnp.float32)

    for g in range(G):
        x = x_refs[g][0]
        k = k_refs[g][0]
        dinv_r = jax.lax.rsqrt(jnp.sum(k, axis=1, keepdims=True))
        dinv_c = jax.lax.rsqrt(jnp.sum(k, axis=0, keepdims=True))
        m1 = jnp.dot(x, w0_ref[...], preferred_element_type=jnp.float32)
        m1s[g * P:g * P + NODES, :] = dinv_r * m1
        dall[g * P:g * P + NODES, :] = dinv_r
        kbd[g * P:g * P + NODES, g * P:g * P + NODES] = k
        asel[g:g + 1, g * P:g * P + NODES] = k[0:1, :] * dinv_c * dinv_r[0:1, :]

    hag = jnp.dot(kbd[...], m1s[...], preferred_element_type=jnp.float32,
                  precision=jax.lax.Precision.HIGHEST)
    h = jnp.maximum(hag * dall[...], 0.0)
    m2 = jnp.dot(h, w1_ref[...], preferred_element_type=jnp.float32)
    w2 = jnp.dot(asel[...], m2, preferred_element_type=jnp.float32,
                 precision=jax.lax.Precision.HIGHEST)
    z = jnp.maximum(w2, 0.0)
    p = jnp.maximum(jnp.dot(z, wlin_ref[...],
                            preferred_element_type=jnp.float32), 0.0)
    out_ref[...] = p.reshape(G, 1, 1)


def kernel(indices, graph_x, kernel, W0, W1, Wlin):
    B = indices.shape[0]
    x_specs = [
        pl.BlockSpec((1, NODES, IN_DIM),
                     (lambda i, idx, g=g: (idx[i * G + g], 0, 0)))
        for g in range(G)
    ]
    k_specs = [
        pl.BlockSpec((1, NODES, NODES),
                     (lambda i, idx, g=g: (idx[i * G + g], 0, 0)))
        for g in range(G)
    ]
    grid_spec = pltpu.PrefetchScalarGridSpec(
        num_scalar_prefetch=1,
        grid=(B // G,),
        in_specs=x_specs + k_specs + [
            pl.BlockSpec((IN_DIM, H0), lambda i, idx: (0, 0)),
            pl.BlockSpec((H0, H1), lambda i, idx: (0, 0)),
            pl.BlockSpec((H1, 1), lambda i, idx: (0, 0)),
        ],
        out_specs=pl.BlockSpec((G, 1, 1), lambda i, idx: (i, 0, 0)),
        scratch_shapes=[
            pltpu.VMEM((GP, GP), jnp.float32),
            pltpu.VMEM((GP, H0), jnp.float32),
            pltpu.VMEM((GP, 1), jnp.float32),
            pltpu.VMEM((G, GP), jnp.float32),
        ],
    )
    out = pl.pallas_call(
        _graph_kernel,
        grid_spec=grid_spec,
        out_shape=jax.ShapeDtypeStruct((B, 1, 1), jnp.float32),
    )(indices, *([graph_x] * G), *([kernel] * G), W0, W1, Wlin)
    return out.reshape(B, 1)
